# Initial kernel scaffold; baseline (speedup 1.0000x reference)
import jax
import jax.numpy as jnp
from jax.experimental import pallas as pl


def kernel(features, cluster_assignments, num_clusters, W1, b1, W2, b2):
    raise NotImplementedError("write your pallas kernel here")


# speedup vs baseline: 7.1988x; 7.1988x over previous
import functools

import jax
import jax.numpy as jnp
from jax import lax
from jax.experimental import pallas as pl
from jax.experimental.pallas import tpu as pltpu
from jax.experimental.pallas import tpu_sc as plsc

B, N, D, C = 8, 16384, 128, 64
T = B * N
NW = 32
TPW = T // NW
K = 256
ACC_W = 144


def _mlp_exp_body(f_ref, w1_ref, b1_ref, w2_ref, b2_ref, o_ref):
    f = f_ref[...]
    h = jnp.maximum(
        jnp.dot(f, w1_ref[...], preferred_element_type=jnp.float32) + b1_ref[...],
        0.0,
    )
    z = jnp.dot(h, w2_ref[...], preferred_element_type=jnp.float32) + b2_ref[...]
    o_ref[...] = jnp.exp(jax.nn.sigmoid(z))


_mlp_call = pl.pallas_call(
    _mlp_exp_body,
    grid=(T // 1024,),
    in_specs=[
        pl.BlockSpec((1024, D), lambda i: (i, 0)),
        pl.BlockSpec((D, D // 2), lambda i: (0, 0)),
        pl.BlockSpec((1, D // 2), lambda i: (0, 0)),
        pl.BlockSpec((D // 2, 1), lambda i: (0, 0)),
        pl.BlockSpec((1, 1), lambda i: (0, 0)),
    ],
    out_specs=pl.BlockSpec((1024, 1), lambda i: (i, 0)),
    out_shape=jax.ShapeDtypeStruct((T, 1), jnp.float32),
)


def _sc_segsum_body(feat_hbm, e_hbm, cid_hbm, out_hbm, fbuf, ebuf, cbuf, acc):
    wid = lax.axis_index("s") * 2 + lax.axis_index("c")
    base = wid * TPW
    zeros = jnp.zeros((16,), jnp.float32)

    def zrow(r, _):
        acc[pl.ds(r * 16, 16)] = zeros
        return 0

    lax.fori_loop(0, C * ACC_W // 16, zrow, 0)

    iota = lax.iota(jnp.int32, 16)

    def chunk_body(ci, _):
        t0 = base + ci * K
        pltpu.sync_copy(feat_hbm.at[pl.ds(t0, K)], fbuf)
        pltpu.sync_copy(e_hbm.at[pl.ds(t0, K)], ebuf)
        pltpu.sync_copy(cid_hbm.at[pl.ds(t0, K)], cbuf)

        def grp_body(g, _):
            g0 = g * 16
            cvec = cbuf[pl.ds(g0, 16)]
            evec = ebuf[pl.ds(g0, 16)]
            for j in range(16):
                base_c = jnp.full((16,), cvec[j] * ACC_W, jnp.int32) + iota
                ev = jnp.full((16,), evec[j], jnp.float32)
                for d in range(8):
                    fv = fbuf[g0 + j, pl.ds(d * 16, 16)]
                    plsc.addupdate_scatter(acc, [base_c + (d * 16)], fv * ev)
                plsc.addupdate_scatter(acc, [base_c + 128], ev)
            return 0

        lax.fori_loop(0, K // 16, grp_body, 0)
        return 0

    lax.fori_loop(0, TPW // K, chunk_body, 0)
    pltpu.sync_copy(acc, out_hbm.at[wid])


_sc_call = functools.partial(
    pl.kernel,
    out_type=jax.ShapeDtypeStruct((NW, C * ACC_W), jnp.float32),
    mesh=plsc.VectorSubcoreMesh(core_axis_name="c", subcore_axis_name="s"),
    compiler_params=pltpu.CompilerParams(needs_layout_passes=False),
    scratch_types=[
        pltpu.VMEM((K, D), jnp.float32),
        pltpu.VMEM((K,), jnp.float32),
        pltpu.VMEM((K,), jnp.int32),
        pltpu.VMEM((C * ACC_W,), jnp.float32),
    ],
)(_sc_segsum_body)


def _combine_body(p_ref, o_ref):
    s = jnp.sum(p_ref[...], axis=0)
    den = s[:, 128:129]
    num = s[:, :128]
    o_ref[...] = jnp.where(den > 0.0, num / den, 0.0)[None]


_combine_call = pl.pallas_call(
    _combine_body,
    grid=(B,),
    in_specs=[pl.BlockSpec((NW // B, C, ACC_W), lambda b: (b, 0, 0))],
    out_specs=pl.BlockSpec((1, C, D), lambda b: (b, 0, 0)),
    out_shape=jax.ShapeDtypeStruct((B, C, D), jnp.float32),
)


def kernel(features, cluster_assignments, num_clusters, W1, b1, W2, b2):
    f_flat = features.reshape(T, D)
    cid_flat = cluster_assignments.reshape(T).astype(jnp.int32)
    e = _mlp_call(f_flat, W1, b1.reshape(1, D // 2), W2, b2.reshape(1, 1))
    p = _sc_call(f_flat, e.reshape(T), cid_flat)
    return _combine_call(p.reshape(NW, C, ACC_W))
